# baseline (speedup 1.0000x reference)
import functools

import jax
import jax.numpy as jnp
from jax import lax
from jax.experimental import pallas as pl
from jax.experimental.pallas import tpu as pltpu
from jax.experimental.pallas import tpu_sc as plsc

EMBEDDING_DIM = 32
BATCH = 16384
NUM_CORES = 2
NUM_SUBCORES = 16
NUM_WORKERS = NUM_CORES * NUM_SUBCORES
B_PER_W = BATCH // NUM_WORKERS
CHUNK = 128
NCHUNK = B_PER_W // CHUNK


@functools.partial(
    pl.kernel,
    mesh=plsc.VectorSubcoreMesh(core_axis_name="c", subcore_axis_name="s"),
    out_type=jax.ShapeDtypeStruct((BATCH, EMBEDDING_DIM), jnp.float32),
    scratch_types=[
        pltpu.VMEM((B_PER_W,), jnp.int32),
        pltpu.VMEM((B_PER_W, EMBEDDING_DIM), jnp.float32),
        pltpu.SemaphoreType.DMA,
    ],
    compiler_params=pltpu.CompilerParams(use_tc_tiling_on_sc=False),
)
def _emb_lookup(table_hbm, idx_hbm, out_hbm, idx_v, rows_v, sem):
    wid = lax.axis_index("s") * NUM_CORES + lax.axis_index("c")
    base = wid * B_PER_W
    pltpu.sync_copy(idx_hbm.at[pl.ds(base, B_PER_W)], idx_v)
    copies = [
        pltpu.async_copy(
            table_hbm.at[idx_v.at[pl.ds(j * CHUNK, CHUNK)]],
            rows_v.at[pl.ds(j * CHUNK, CHUNK), :],
            sem,
        )
        for j in range(NCHUNK)
    ]
    for c in copies:
        c.wait()
    pltpu.sync_copy(rows_v, out_hbm.at[pl.ds(base, B_PER_W)])


def kernel(x, weight):
    return _emb_lookup(weight, x.astype(jnp.int32))


# speedup vs baseline: 1.5167x; 1.5167x over previous
import functools

import jax
import jax.numpy as jnp
from jax import lax
from jax.experimental import pallas as pl
from jax.experimental.pallas import tpu as pltpu
from jax.experimental.pallas import tpu_sc as plsc

NUM_EMB = 1_000_000
EMBEDDING_DIM = 32
BATCH = 16384
NUM_CORES = 2
NUM_SUBCORES = 16
NUM_WORKERS = NUM_CORES * NUM_SUBCORES
B_PER_W = BATCH // NUM_WORKERS
CHUNK = 32
NCHUNK = B_PER_W // CHUNK
LANES = 16


@functools.partial(
    pl.kernel,
    mesh=plsc.VectorSubcoreMesh(core_axis_name="c", subcore_axis_name="s"),
    out_type=jax.ShapeDtypeStruct((BATCH, EMBEDDING_DIM), jnp.float32),
    scratch_types=[
        pltpu.VMEM((B_PER_W,), jnp.int32),
        pltpu.VMEM((CHUNK, 8, EMBEDDING_DIM), jnp.float32),
        pltpu.VMEM((CHUNK, 8, EMBEDDING_DIM), jnp.float32),
        pltpu.VMEM((CHUNK, EMBEDDING_DIM), jnp.float32),
        pltpu.SemaphoreType.DMA,
        pltpu.SemaphoreType.DMA,
    ],
    compiler_params=pltpu.CompilerParams(
        use_tc_tiling_on_sc=True, needs_layout_passes=False
    ),
)
def _emb_lookup(tbl, idx_hbm, out, idx_v, buf0, buf1, rows_v, sem0, sem1):
    wid = lax.axis_index("s") * NUM_CORES + lax.axis_index("c")
    base = wid * B_PER_W
    pltpu.sync_copy(idx_hbm.at[pl.ds(base, B_PER_W)], idx_v)
    t3 = tbl.reshape(NUM_EMB // 8, 8, EMBEDDING_DIM)
    bufs = (buf0, buf1)
    sems = (sem0, sem1)

    def fire(c, buf, sem):
        for g in range(CHUNK // LANES):
            ivec = idx_v[pl.ds(c * CHUNK + g * LANES, LANES)]
            tvec = lax.shift_right_logical(ivec, 3)
            for kk in range(LANES):
                pltpu.async_copy(t3.at[tvec[kk]], buf.at[g * LANES + kk], sem)

    def drain(buf, sem):
        pltpu.make_async_copy(t3.at[pl.ds(0, CHUNK)], buf, sem).wait()

    def extract(c, buf):
        for g in range(CHUNK // LANES):
            ivec = idx_v[pl.ds(c * CHUNK + g * LANES, LANES)]
            rvec = lax.bitwise_and(ivec, 7)
            for kk in range(LANES):
                j = g * LANES + kk
                for dd in range(EMBEDDING_DIM // LANES):
                    cols = lax.iota(jnp.int32, LANES) + dd * LANES
                    vals = plsc.load_gather(
                        buf,
                        [jnp.full((LANES,), j, jnp.int32),
                         jnp.broadcast_to(rvec[kk], (LANES,)),
                         cols],
                    )
                    rows_v[j, pl.ds(dd * LANES, LANES)] = vals

    fire(0, bufs[0], sems[0])
    for c in range(NCHUNK):
        if c + 1 < NCHUNK:
            fire(c + 1, bufs[(c + 1) % 2], sems[(c + 1) % 2])
        drain(bufs[c % 2], sems[c % 2])
        extract(c, bufs[c % 2])
        pltpu.sync_copy(rows_v, out.at[pl.ds(base + c * CHUNK, CHUNK)])


def kernel(x, weight):
    return _emb_lookup(weight, x.astype(jnp.int32))
